# baseline (speedup 1.0000x reference)
import functools
import math

import jax
import jax.numpy as jnp
from jax import lax
from jax.experimental import pallas as pl
from jax.experimental.pallas import tpu as pltpu
from jax.experimental.pallas import tpu_sc as plsc

L = 16
NW = 32
NPW = 320
CHUNK = 80
DEG = 32


def _context_body(agg_hbm, nb_hbm, ctx_hbm, aggv, idxv, rows, ctxv, sem):
    cid = lax.axis_index("c")
    sid = lax.axis_index("s")
    wid = sid * 2 + cid
    node_base = wid * NPW
    iota = lax.broadcasted_iota(jnp.int32, (L,), 0)

    for chunk in range(NPW // CHUNK):
        nbase = node_base + chunk * CHUNK
        pltpu.sync_copy(agg_hbm.at[pl.ds(nbase, CHUNK)], aggv)
        pltpu.sync_copy(nb_hbm.at[pl.ds(nbase * DEG, CHUNK * DEG)], idxv)
        pltpu.async_copy(agg_hbm.at[idxv], rows, sem).wait()

        def node_body(t, carry):
            base = t * DEG
            r0 = iota + base
            r1 = iota + (base + L)
            av = aggv[t, :]
            sa = jnp.zeros((L,), jnp.float32)
            sb = jnp.zeros((L,), jnp.float32)
            for dd in range(L):
                a_d = av[dd]
                col = jnp.full((L,), dd, jnp.int32)
                sa = sa + a_d * plsc.load_gather(rows, [r0, col])
                sb = sb + a_d * plsc.load_gather(rows, [r1, col])
            sa = sa * jnp.float32(1.0 / math.sqrt(L))
            sb = sb * jnp.float32(1.0 / math.sqrt(L))
            mx = jnp.maximum(jnp.max(sa), jnp.max(sb))
            ea = jnp.exp(sa - mx)
            eb = jnp.exp(sb - mx)
            total = jnp.sum(ea) + jnp.sum(eb)
            inv = jnp.ones((L,), jnp.float32) / jnp.broadcast_to(total, (L,))
            pa = ea * inv
            pb = eb * inv
            ctx = jnp.zeros((L,), jnp.float32)
            for m in range(L):
                ctx = ctx + pa[m] * rows[base + m, :]
            for m in range(L):
                ctx = ctx + pb[m] * rows[base + L + m, :]
            ctxv[t, :] = ctx
            return carry

        lax.fori_loop(0, CHUNK, node_body, 0)
        pltpu.sync_copy(ctxv, ctx_hbm.at[pl.ds(nbase, CHUNK)])


def _context_sc(agg_pad, nb_pad, node_pad):
    f = functools.partial(
        pl.kernel,
        out_type=jax.ShapeDtypeStruct((node_pad, L), jnp.float32),
        mesh=plsc.VectorSubcoreMesh(core_axis_name="c", subcore_axis_name="s"),
        compiler_params=pltpu.CompilerParams(
            needs_layout_passes=False, use_tc_tiling_on_sc=False),
        scratch_types=[
            pltpu.VMEM((CHUNK, L), jnp.float32),
            pltpu.VMEM((CHUNK * DEG,), jnp.int32),
            pltpu.VMEM((CHUNK * DEG, L), jnp.float32),
            pltpu.VMEM((CHUNK, L), jnp.float32),
            pltpu.SemaphoreType.DMA,
        ],
    )(_context_body)
    return f(agg_pad, nb_pad)


def _adj_body(a_ref, b_ref, o_ref):
    x = lax.dot_general(
        a_ref[...], b_ref[...], (((1,), (1,)), ((), ())),
        preferred_element_type=jnp.float32)
    o_ref[...] = 0.5 * (jnp.tanh(0.5 * x) + 1.0)


def _attr_body(nf_ref, ctx_ref, w1_ref, w2_ref, b_ref, o_ref):
    acc = jnp.dot(nf_ref[...], w1_ref[...], preferred_element_type=jnp.float32)
    acc = acc + jnp.dot(ctx_ref[...], w2_ref[...], preferred_element_type=jnp.float32)
    o_ref[...] = jax.nn.sigmoid(acc + b_ref[...])


def kernel(node_feature, agg_feature, nb_id, W, b):
    n = agg_feature.shape[0]
    d = agg_feature.shape[1]
    node_dim = node_feature.shape[1]
    node_pad = NW * NPW

    agg_pad = jnp.pad(agg_feature, ((0, node_pad - n), (0, 0)))
    nb_pad = jnp.pad(nb_id, (0, node_pad * DEG - nb_id.shape[0]))

    ctx = _context_sc(agg_pad, nb_pad, node_pad)[:n]

    BI, BJ = 1024, 4096
    adj = pl.pallas_call(
        _adj_body,
        grid=(pl.cdiv(n, BI), pl.cdiv(n, BJ)),
        in_specs=[
            pl.BlockSpec((BI, d), lambda i, j: (i, 0)),
            pl.BlockSpec((BJ, d), lambda i, j: (j, 0)),
        ],
        out_specs=pl.BlockSpec((BI, BJ), lambda i, j: (i, j)),
        out_shape=jax.ShapeDtypeStruct((n, n), jnp.float32),
    )(agg_feature, agg_feature)

    BR = 1024
    attr = pl.pallas_call(
        _attr_body,
        grid=(pl.cdiv(n, BR),),
        in_specs=[
            pl.BlockSpec((BR, node_dim), lambda i: (i, 0)),
            pl.BlockSpec((BR, d), lambda i: (i, 0)),
            pl.BlockSpec((node_dim, node_dim), lambda i: (0, 0)),
            pl.BlockSpec((d, node_dim), lambda i: (0, 0)),
            pl.BlockSpec((1, node_dim), lambda i: (0, 0)),
        ],
        out_specs=pl.BlockSpec((BR, node_dim), lambda i: (i, 0)),
        out_shape=jax.ShapeDtypeStruct((n, node_dim), jnp.float32),
    )(node_feature, ctx, W[:node_dim], W[node_dim:], b.reshape(1, node_dim))

    return (attr, adj)


# speedup vs baseline: 1.2879x; 1.2879x over previous
import functools
import math

import jax
import jax.numpy as jnp
from jax import lax
from jax.experimental import pallas as pl
from jax.experimental.pallas import tpu as pltpu
from jax.experimental.pallas import tpu_sc as plsc

L = 16
NW = 32
NPW = 312
CHUNK = 78
TAIL = 16
DEG = 32


def _context_chunk(agg_hbm, nb_hbm, ctx_hbm, aggv, idxv, rows, ctxv, sem,
                   nbase, count, iota):
    pltpu.sync_copy(agg_hbm.at[pl.ds(nbase, count)], aggv.at[pl.ds(0, count)])
    pltpu.sync_copy(nb_hbm.at[pl.ds(nbase * DEG, count * DEG)],
                    idxv.at[pl.ds(0, count * DEG)])
    pltpu.async_copy(agg_hbm.at[idxv.at[pl.ds(0, count * DEG)]],
                     rows.at[pl.ds(0, count * DEG)], sem).wait()

    def node_body(t, carry):
        base = t * DEG
        r0 = iota + base
        r1 = iota + (base + L)
        av = aggv[t, :]
        sa = jnp.zeros((L,), jnp.float32)
        sb = jnp.zeros((L,), jnp.float32)
        for dd in range(L):
            a_d = av[dd]
            col = jnp.full((L,), dd, jnp.int32)
            sa = sa + a_d * plsc.load_gather(rows, [r0, col])
            sb = sb + a_d * plsc.load_gather(rows, [r1, col])
        sa = sa * jnp.float32(1.0 / math.sqrt(L))
        sb = sb * jnp.float32(1.0 / math.sqrt(L))
        mx = jnp.maximum(jnp.max(sa), jnp.max(sb))
        ea = jnp.exp(sa - mx)
        eb = jnp.exp(sb - mx)
        total = jnp.sum(ea) + jnp.sum(eb)
        inv = jnp.ones((L,), jnp.float32) / jnp.broadcast_to(total, (L,))
        pa = ea * inv
        pb = eb * inv
        ctx = jnp.zeros((L,), jnp.float32)
        for m in range(L):
            ctx = ctx + pa[m] * rows[base + m, :]
        for m in range(L):
            ctx = ctx + pb[m] * rows[base + L + m, :]
        ctxv[t, :] = ctx
        return carry

    lax.fori_loop(0, count, node_body, 0)
    pltpu.sync_copy(ctxv.at[pl.ds(0, count)], ctx_hbm.at[pl.ds(nbase, count)])


def _context_body(agg_hbm, nb_hbm, ctx_hbm, aggv, idxv, rows, ctxv, sem):
    cid = lax.axis_index("c")
    sid = lax.axis_index("s")
    wid = sid * 2 + cid
    node_base = wid * NPW
    iota = lax.broadcasted_iota(jnp.int32, (L,), 0)

    for chunk in range(NPW // CHUNK):
        _context_chunk(agg_hbm, nb_hbm, ctx_hbm, aggv, idxv, rows, ctxv, sem,
                       node_base + chunk * CHUNK, CHUNK, iota)

    @pl.when(wid == NW - 1)
    def _tail():
        _context_chunk(agg_hbm, nb_hbm, ctx_hbm, aggv, idxv, rows, ctxv, sem,
                       NW * NPW, TAIL, iota)


def _context_sc(agg_feature, nb_id, n):
    f = functools.partial(
        pl.kernel,
        out_type=jax.ShapeDtypeStruct((n, L), jnp.float32),
        mesh=plsc.VectorSubcoreMesh(core_axis_name="c", subcore_axis_name="s"),
        compiler_params=pltpu.CompilerParams(
            needs_layout_passes=False, use_tc_tiling_on_sc=False),
        scratch_types=[
            pltpu.VMEM((CHUNK, L), jnp.float32),
            pltpu.VMEM((CHUNK * DEG,), jnp.int32),
            pltpu.VMEM((CHUNK * DEG, L), jnp.float32),
            pltpu.VMEM((CHUNK, L), jnp.float32),
            pltpu.SemaphoreType.DMA,
        ],
    )(_context_body)
    return f(agg_feature, nb_id)


def _adj_body(a_ref, b_ref, o_ref):
    x = lax.dot_general(
        a_ref[...], b_ref[...], (((1,), (1,)), ((), ())),
        preferred_element_type=jnp.float32)
    o_ref[...] = 0.5 * (jnp.tanh(0.5 * x) + 1.0)


def _attr_body(nf_ref, ctx_ref, w1_ref, w2_ref, b_ref, o_ref):
    acc = jnp.dot(nf_ref[...], w1_ref[...], preferred_element_type=jnp.float32)
    acc = acc + jnp.dot(ctx_ref[...], w2_ref[...], preferred_element_type=jnp.float32)
    o_ref[...] = jax.nn.sigmoid(acc + b_ref[...])


def kernel(node_feature, agg_feature, nb_id, W, b):
    n = agg_feature.shape[0]
    d = agg_feature.shape[1]
    node_dim = node_feature.shape[1]

    ctx = _context_sc(agg_feature, nb_id, n)

    BI, BJ = 2048, 2048
    adj = pl.pallas_call(
        _adj_body,
        grid=(pl.cdiv(n, BI), pl.cdiv(n, BJ)),
        in_specs=[
            pl.BlockSpec((BI, d), lambda i, j: (i, 0)),
            pl.BlockSpec((BJ, d), lambda i, j: (j, 0)),
        ],
        out_specs=pl.BlockSpec((BI, BJ), lambda i, j: (i, j)),
        out_shape=jax.ShapeDtypeStruct((n, n), jnp.float32),
    )(agg_feature, agg_feature)

    BR = 2048
    attr = pl.pallas_call(
        _attr_body,
        grid=(pl.cdiv(n, BR),),
        in_specs=[
            pl.BlockSpec((BR, node_dim), lambda i: (i, 0)),
            pl.BlockSpec((BR, d), lambda i: (i, 0)),
            pl.BlockSpec((node_dim, node_dim), lambda i: (0, 0)),
            pl.BlockSpec((d, node_dim), lambda i: (0, 0)),
            pl.BlockSpec((1, node_dim), lambda i: (0, 0)),
        ],
        out_specs=pl.BlockSpec((BR, node_dim), lambda i: (i, 0)),
        out_shape=jax.ShapeDtypeStruct((n, node_dim), jnp.float32),
    )(node_feature, ctx, W[:node_dim], W[node_dim:], b.reshape(1, node_dim))

    return (attr, adj)


# speedup vs baseline: 1.3100x; 1.0172x over previous
import functools
import math

import jax
import jax.numpy as jnp
from jax import lax
from jax.experimental import pallas as pl
from jax.experimental.pallas import tpu as pltpu
from jax.experimental.pallas import tpu_sc as plsc

L = 16
NW = 32
NPW = 312
CHUNK = 78
TAIL = 16
DEG = 32


def _context_chunk(agg_hbm, nb_hbm, ctx_hbm, aggv, idxv, rows, ctxv, sem,
                   nbase, count, iota):
    pltpu.sync_copy(agg_hbm.at[pl.ds(nbase, count)], aggv.at[pl.ds(0, count)])
    pltpu.sync_copy(nb_hbm.at[pl.ds(nbase * DEG, count * DEG)],
                    idxv.at[pl.ds(0, count * DEG)])
    pltpu.async_copy(agg_hbm.at[idxv.at[pl.ds(0, count * DEG)]],
                     rows.at[pl.ds(0, count * DEG)], sem).wait()

    def node_body(t, carry):
        base = t * DEG
        r0 = iota + base
        r1 = iota + (base + L)
        av = aggv[t, :]
        sa = jnp.zeros((L,), jnp.float32)
        sb = jnp.zeros((L,), jnp.float32)
        for dd in range(L):
            a_d = av[dd]
            col = jnp.full((L,), dd, jnp.int32)
            sa = sa + a_d * plsc.load_gather(rows, [r0, col])
            sb = sb + a_d * plsc.load_gather(rows, [r1, col])
        sa = sa * jnp.float32(1.0 / math.sqrt(L))
        sb = sb * jnp.float32(1.0 / math.sqrt(L))
        mx = jnp.maximum(jnp.max(sa), jnp.max(sb))
        ea = jnp.exp(sa - mx)
        eb = jnp.exp(sb - mx)
        total = jnp.sum(ea) + jnp.sum(eb)
        inv = jnp.ones((L,), jnp.float32) / jnp.broadcast_to(total, (L,))
        pa = ea * inv
        pb = eb * inv
        ctx = jnp.zeros((L,), jnp.float32)
        for m in range(L):
            ctx = ctx + pa[m] * rows[base + m, :]
        for m in range(L):
            ctx = ctx + pb[m] * rows[base + L + m, :]
        ctxv[t, :] = ctx
        return carry

    lax.fori_loop(0, count, node_body, 0)
    pltpu.sync_copy(ctxv.at[pl.ds(0, count)], ctx_hbm.at[pl.ds(nbase, count)])


def _context_body(agg_hbm, nb_hbm, ctx_hbm, aggv, idxv, rows, ctxv, sem):
    cid = lax.axis_index("c")
    sid = lax.axis_index("s")
    wid = sid * 2 + cid
    node_base = wid * NPW
    iota = lax.broadcasted_iota(jnp.int32, (L,), 0)

    for chunk in range(NPW // CHUNK):
        _context_chunk(agg_hbm, nb_hbm, ctx_hbm, aggv, idxv, rows, ctxv, sem,
                       node_base + chunk * CHUNK, CHUNK, iota)

    @pl.when(wid == NW - 1)
    def _tail():
        _context_chunk(agg_hbm, nb_hbm, ctx_hbm, aggv, idxv, rows, ctxv, sem,
                       NW * NPW, TAIL, iota)


def _context_sc(agg_feature, nb_id, n):
    f = functools.partial(
        pl.kernel,
        out_type=jax.ShapeDtypeStruct((n, L), jnp.float32),
        mesh=plsc.VectorSubcoreMesh(core_axis_name="c", subcore_axis_name="s"),
        compiler_params=pltpu.CompilerParams(
            needs_layout_passes=False, use_tc_tiling_on_sc=False),
        scratch_types=[
            pltpu.VMEM((CHUNK, L), jnp.float32),
            pltpu.VMEM((CHUNK * DEG,), jnp.int32),
            pltpu.VMEM((CHUNK * DEG, L), jnp.float32),
            pltpu.VMEM((CHUNK, L), jnp.float32),
            pltpu.SemaphoreType.DMA,
        ],
    )(_context_body)
    return f(agg_feature, nb_id)


def _adj_body(a_ref, b_ref, o_ref):
    x = lax.dot_general(
        a_ref[...], b_ref[...], (((1,), (1,)), ((), ())),
        preferred_element_type=jnp.float32)
    o_ref[...] = 0.5 * (jnp.tanh(0.5 * x) + 1.0)


def _attr_body(nf_ref, ctx_ref, w1_ref, w2_ref, b_ref, o_ref):
    acc = jnp.dot(nf_ref[...], w1_ref[...], preferred_element_type=jnp.float32)
    acc = acc + jnp.dot(ctx_ref[...], w2_ref[...], preferred_element_type=jnp.float32)
    o_ref[...] = jax.nn.sigmoid(acc + b_ref[...])


def kernel(node_feature, agg_feature, nb_id, W, b):
    n = agg_feature.shape[0]
    d = agg_feature.shape[1]
    node_dim = node_feature.shape[1]

    ctx = _context_sc(agg_feature, nb_id, n)

    BI, BJ = 2560, 2560
    adj = pl.pallas_call(
        _adj_body,
        grid=(pl.cdiv(n, BI), pl.cdiv(n, BJ)),
        in_specs=[
            pl.BlockSpec((BI, d), lambda i, j: (i, 0)),
            pl.BlockSpec((BJ, d), lambda i, j: (j, 0)),
        ],
        out_specs=pl.BlockSpec((BI, BJ), lambda i, j: (i, j)),
        out_shape=jax.ShapeDtypeStruct((n, n), jnp.float32),
    )(agg_feature, agg_feature)

    BR = 2048
    attr = pl.pallas_call(
        _attr_body,
        grid=(pl.cdiv(n, BR),),
        in_specs=[
            pl.BlockSpec((BR, node_dim), lambda i: (i, 0)),
            pl.BlockSpec((BR, d), lambda i: (i, 0)),
            pl.BlockSpec((node_dim, node_dim), lambda i: (0, 0)),
            pl.BlockSpec((d, node_dim), lambda i: (0, 0)),
            pl.BlockSpec((1, node_dim), lambda i: (0, 0)),
        ],
        out_specs=pl.BlockSpec((BR, node_dim), lambda i: (i, 0)),
        out_shape=jax.ShapeDtypeStruct((n, node_dim), jnp.float32),
    )(node_feature, ctx, W[:node_dim], W[node_dim:], b.reshape(1, node_dim))

    return (attr, adj)


# speedup vs baseline: 1.4229x; 1.0862x over previous
import functools
import math

import jax
import jax.numpy as jnp
from jax import lax
from jax.experimental import pallas as pl
from jax.experimental.pallas import tpu as pltpu
from jax.experimental.pallas import tpu_sc as plsc

L = 16
NW = 32
NPW = 312
CHUNK = 104
TAIL = 16
DEG = 32


def _context_chunk(agg_hbm, nb_hbm, ctx_hbm, aggv, idxv, rows, ctxv, sem,
                   nbase, count, iota):
    pltpu.sync_copy(agg_hbm.at[pl.ds(nbase, count)], aggv.at[pl.ds(0, count)])
    pltpu.sync_copy(nb_hbm.at[pl.ds(nbase * DEG, count * DEG)],
                    idxv.at[pl.ds(0, count * DEG)])
    pltpu.async_copy(agg_hbm.at[idxv.at[pl.ds(0, count * DEG)]],
                     rows.at[pl.ds(0, count * DEG)], sem).wait()

    def node_body(t, carry):
        base = t * DEG
        r0 = iota + base
        r1 = iota + (base + L)
        av = aggv[t, :]
        sa = jnp.zeros((L,), jnp.float32)
        sb = jnp.zeros((L,), jnp.float32)
        for dd in range(L):
            a_d = av[dd]
            col = jnp.full((L,), dd, jnp.int32)
            sa = sa + a_d * plsc.load_gather(rows, [r0, col])
            sb = sb + a_d * plsc.load_gather(rows, [r1, col])
        sa = sa * jnp.float32(1.0 / math.sqrt(L))
        sb = sb * jnp.float32(1.0 / math.sqrt(L))
        mx = jnp.maximum(jnp.max(sa), jnp.max(sb))
        ea = jnp.exp(sa - mx)
        eb = jnp.exp(sb - mx)
        total = jnp.sum(ea) + jnp.sum(eb)
        inv = jnp.ones((L,), jnp.float32) / jnp.broadcast_to(total, (L,))
        pa = ea * inv
        pb = eb * inv
        ctx = jnp.zeros((L,), jnp.float32)
        for m in range(L):
            ctx = ctx + pa[m] * rows[base + m, :]
        for m in range(L):
            ctx = ctx + pb[m] * rows[base + L + m, :]
        ctxv[t >> 3, pl.ds((t & 7) * L, L)] = ctx
        return carry

    lax.fori_loop(0, count, node_body, 0)
    rows_cnt = count * L // 128
    pltpu.sync_copy(ctxv.at[pl.ds(0, rows_cnt)],
                    ctx_hbm.at[pl.ds(nbase * L // 128, rows_cnt)])


def _context_body(agg_hbm, nb_hbm, ctx_hbm, aggv, idxv, rows, ctxv, sem):
    cid = lax.axis_index("c")
    sid = lax.axis_index("s")
    wid = sid * 2 + cid
    node_base = wid * NPW
    iota = lax.broadcasted_iota(jnp.int32, (L,), 0)

    for chunk in range(NPW // CHUNK):
        _context_chunk(agg_hbm, nb_hbm, ctx_hbm, aggv, idxv, rows, ctxv, sem,
                       node_base + chunk * CHUNK, CHUNK, iota)

    @pl.when(wid == NW - 1)
    def _tail():
        _context_chunk(agg_hbm, nb_hbm, ctx_hbm, aggv, idxv, rows, ctxv, sem,
                       NW * NPW, TAIL, iota)


def _context_sc(agg_feature, nb_id, n):
    f = functools.partial(
        pl.kernel,
        out_type=jax.ShapeDtypeStruct((n * L // 128, 128), jnp.float32),
        mesh=plsc.VectorSubcoreMesh(core_axis_name="c", subcore_axis_name="s"),
        compiler_params=pltpu.CompilerParams(
            needs_layout_passes=False, use_tc_tiling_on_sc=False),
        scratch_types=[
            pltpu.VMEM((CHUNK, L), jnp.float32),
            pltpu.VMEM((CHUNK * DEG,), jnp.int32),
            pltpu.VMEM((CHUNK * DEG, L), jnp.float32),
            pltpu.VMEM((CHUNK * L // 128, 128), jnp.float32),
            pltpu.SemaphoreType.DMA,
        ],
    )(_context_body)
    return f(agg_feature, nb_id)


def _adj_body(a_ref, b_ref, o_ref):
    x = lax.dot_general(
        a_ref[...], b_ref[...], (((0,), (0,)), ((), ())),
        preferred_element_type=jnp.float32)
    o_ref[...] = 0.5 * (jnp.tanh(0.5 * x) + 1.0)


def _attr_body(nf_ref, ctx_ref, w_ref, b_ref, o_ref):
    nd = nf_ref.shape[1]
    w1 = w_ref[0:nd, :]
    w2 = w_ref[nd:, :]
    acc = jnp.dot(nf_ref[...], w1, preferred_element_type=jnp.float32)
    w2t = jnp.concatenate([w2] * 8, axis=0)
    jrow = lax.broadcasted_iota(jnp.int32, (128, 128), 0)
    parts = [jnp.where(jnp.right_shift(jrow, 4) == blk, w2t, 0.0)
             for blk in range(8)]
    w2big = jnp.concatenate(parts, axis=1)
    y2p = jnp.dot(ctx_ref[...], w2big, preferred_element_type=jnp.float32)
    y2 = jnp.reshape(y2p, (o_ref.shape[0], o_ref.shape[1]))
    o_ref[...] = jax.nn.sigmoid(acc + y2 + b_ref[...])


def kernel(node_feature, agg_feature, nb_id, W, b):
    n = agg_feature.shape[0]
    d = agg_feature.shape[1]
    node_dim = node_feature.shape[1]

    ctx = _context_sc(agg_feature, nb_id, n)

    aggT = agg_feature.T
    BI, BJ = 2560, 2560
    adj = pl.pallas_call(
        _adj_body,
        grid=(pl.cdiv(n, BI), pl.cdiv(n, BJ)),
        in_specs=[
            pl.BlockSpec((d, BI), lambda i, j: (0, i)),
            pl.BlockSpec((d, BJ), lambda i, j: (0, j)),
        ],
        out_specs=pl.BlockSpec((BI, BJ), lambda i, j: (i, j)),
        out_shape=jax.ShapeDtypeStruct((n, n), jnp.float32),
    )(aggT, aggT)

    BR = 2560
    attr = pl.pallas_call(
        _attr_body,
        grid=(pl.cdiv(n, BR),),
        in_specs=[
            pl.BlockSpec((BR, node_dim), lambda i: (i, 0)),
            pl.BlockSpec((BR * d // 128, 128), lambda i: (i, 0)),
            pl.BlockSpec((node_dim + d, node_dim), lambda i: (0, 0)),
            pl.BlockSpec((1, node_dim), lambda i: (0, 0)),
        ],
        out_specs=pl.BlockSpec((BR, node_dim), lambda i: (i, 0)),
        out_shape=jax.ShapeDtypeStruct((n, node_dim), jnp.float32),
    )(node_feature, ctx, W, b.reshape(1, node_dim))

    return (attr, adj)
